# baseline (speedup 1.0000x reference)
import functools

import jax
import jax.numpy as jnp
from jax import lax
from jax.experimental import pallas as pl
from jax.experimental.pallas import tpu as pltpu
from jax.experimental.pallas import tpu_sc as plsc

U = 256
N = 1024
P = 256
C = 256
D = 128
OUTD = 128
NC, NS, L = 2, 16, 16
NW = NC * NS
BPT = U // NW
CH = 32
CH_SHIFT = 5

_mesh = plsc.VectorSubcoreMesh(core_axis_name="c", subcore_axis_name="s")


@functools.partial(
    pl.kernel,
    out_type=(jax.ShapeDtypeStruct((U, C), jnp.float32),
              jax.ShapeDtypeStruct((U, D), jnp.float32)),
    mesh=_mesh,
    scratch_types=[
        pltpu.VMEM((6 * U,), jnp.float32),
        pltpu.VMEM((N,), jnp.float32),
        pltpu.VMEM((N,), jnp.float32),
        pltpu.VMEM((N,), jnp.float32),
        pltpu.VMEM((P,), jnp.float32),
        pltpu.VMEM((P,), jnp.float32),
        pltpu.VMEM((P,), jnp.float32),
        pltpu.VMEM((N,), jnp.int32),
        pltpu.VMEM((CH, C), jnp.float32),
        pltpu.VMEM((CH, D), jnp.float32),
        pltpu.VMEM((BPT, C), jnp.float32),
        pltpu.VMEM((BPT, D), jnp.float32),
        pltpu.SemaphoreType.DMA,
    ],
    compiler_params=pltpu.CompilerParams(needs_layout_passes=False),
)
def _sc_pool(ub_hbm, sx_hbm, sy_hbm, sz_hbm, ax_hbm, ay_hbm, az_hbm,
             sft_hbm, bft_hbm,
             g1_hbm, g2_hbm,
             ub_v, sx_v, sy_v, sz_v, ax_v, ay_v, az_v,
             idx_v, rows1_v, rows2_v, g1_v, g2_v, sem):
    wid = lax.axis_index("s") * NC + lax.axis_index("c")
    u_base = wid * BPT

    pltpu.sync_copy(ub_hbm, ub_v)
    pltpu.sync_copy(sx_hbm, sx_v)
    pltpu.sync_copy(sy_hbm, sy_v)
    pltpu.sync_copy(sz_hbm, sz_v)
    pltpu.sync_copy(ax_hbm, ax_v)
    pltpu.sync_copy(ay_hbm, ay_v)
    pltpu.sync_copy(az_hbm, az_v)

    def pool(u, lb, npts, xr, yr, zr, table_hbm, rows_ref, nchan, out_ref):
        def bcast(r):
            return plsc.load_gather(
                ub_v, [jnp.full((L,), r * U + u, jnp.int32)])
        cx, cy, cz = bcast(0), bcast(1), bcast(2)
        hx, hy, hz = bcast(3) * 0.5, bcast(4) * 0.5, bcast(5) * 0.5
        lox, hix = cx - hx, cx + hx
        loy, hiy = cy - hy, cy + hy
        loz, hiz = cz - hz, cz + hz
        nvec = nchan // L
        nv = npts // L

        def pf(i, c):
            idx_v[pl.ds(i * L, L)] = jnp.full((L,), npts, jnp.int32)
            return c
        lax.fori_loop(0, nv, pf, jnp.int32(0))

        def mk(i, cnt):
            xv = xr[pl.ds(i * L, L)]
            yv = yr[pl.ds(i * L, L)]
            zv = zr[pl.ds(i * L, L)]
            m = ((xv >= lox) & (xv <= hix) & (yv >= loy) & (yv <= hiy)
                 & (zv >= loz) & (zv <= hiz))
            mi = m.astype(jnp.int32)
            cs = plsc.cumsum(mi)
            pos = (cnt + cs) - mi
            ids = lax.iota(jnp.int32, L) + i * L
            plsc.store_scatter(idx_v, [pos], ids, mask=m)
            return cnt + jnp.max(cs)
        cnt = lax.fori_loop(0, nv, mk, jnp.int32(0))

        nch = (cnt + (CH - 1)) >> CH_SHIFT
        accs = tuple(jnp.full((L,), -jnp.inf, jnp.float32)
                     for _ in range(nvec))

        def ch(k, accs):
            cp = pltpu.async_copy(
                table_hbm.at[idx_v.at[pl.ds(k * CH, CH)]], rows_ref, sem)
            cp.wait()

            def rr(r, accs):
                return tuple(
                    jnp.maximum(accs[j], rows_ref[r, pl.ds(j * L, L)])
                    for j in range(nvec))
            return lax.fori_loop(0, CH, rr, accs)
        accs = lax.fori_loop(0, nch, ch, accs)

        fix = jnp.where(jnp.full((L,), cnt, jnp.int32) == npts,
                        jnp.full((L,), -jnp.inf, jnp.float32),
                        jnp.zeros((L,), jnp.float32))
        for j in range(nvec):
            out_ref[lb, pl.ds(j * L, L)] = jnp.maximum(accs[j], fix)

    for lb in range(BPT):
        u = u_base + lb
        pool(u, lb, N, sx_v, sy_v, sz_v, sft_hbm, rows1_v, C, g1_v)
        pool(u, lb, P, ax_v, ay_v, az_v, bft_hbm, rows2_v, D, g2_v)

    pltpu.sync_copy(g1_v, g1_hbm.at[pl.ds(u_base, BPT)])
    pltpu.sync_copy(g2_v, g2_hbm.at[pl.ds(u_base, BPT)])


def _mm_body(x_ref, w_ref, b_ref, out_ref):
    out = lax.dot_general(x_ref[:], w_ref[:], (((1,), (1,)), ((), ())),
                          preferred_element_type=jnp.float32)
    out = out + b_ref[:]
    out_ref[:] = jax.nn.sigmoid(jnp.log(jnp.abs(out + 1e-6)))


def kernel(union_box, box_features, agg_xyz, seed_feature, seed_xyz,
           box_feature_union, W, b):
    f32 = jnp.float32
    ub6 = union_box[0].T.reshape(-1)
    sx, sy, sz = (seed_xyz[:, k] for k in range(3))
    ax, ay, az = (agg_xyz[:, k] for k in range(3))
    sft = jnp.concatenate(
        [seed_feature.T, jnp.full((8, C), -jnp.inf, f32)], axis=0)
    bft = jnp.concatenate(
        [box_features, jnp.full((8, D), -jnp.inf, f32)], axis=0)

    g1, g2 = _sc_pool(ub6, sx, sy, sz, ax, ay, az, sft, bft)
    g1 = g1.astype(jnp.float16).astype(f32)
    g2 = g2.astype(jnp.float16).astype(f32)
    glob = jnp.concatenate([g1, g2, box_feature_union[:, 0, :]], axis=1)

    return pl.pallas_call(
        _mm_body,
        out_shape=jax.ShapeDtypeStruct((U, OUTD), jnp.float32),
    )(glob, W, b.reshape(1, OUTD))
